# Initial kernel scaffold; baseline (speedup 1.0000x reference)
import functools

import jax
import jax.numpy as jnp
from jax import lax
from jax.experimental import pallas as pl
from jax.experimental.pallas import tpu as pltpu
from jax.experimental.pallas import tpu_sc as plsc

N = 10000
E = 320000
D = 128
D_OUT = 64

NC = 2
NS = 16
NW = NC * NS
EP = E // NW
CH = 80
NCHUNK = EP // CH
RP = N // NS

_sc_mesh = plsc.VectorSubcoreMesh(core_axis_name="c", subcore_axis_name="s")


@functools.partial(
    pl.kernel,
    out_type=jax.ShapeDtypeStruct((NC * N, 8), jnp.float32),
    mesh=_sc_mesh,
    scratch_types=[
        pltpu.VMEM((RP, 8), jnp.float32),
        pltpu.VMEM((CH, 8), jnp.float32),
        pltpu.VMEM((1, CH), jnp.int32),
        pltpu.VMEM_SHARED((N, 8), jnp.float32),
    ],
)
def _sc_degree(dst_hbm, zeros_hbm, ones_hbm, out_hbm, stage, ones_v, didx, acc):
    c = lax.axis_index("c")
    s = lax.axis_index("s")
    wid = c * NS + s
    pltpu.sync_copy(zeros_hbm, stage)
    pltpu.sync_copy(stage, acc.at[pl.ds(s * RP, RP)])
    pltpu.sync_copy(ones_hbm, ones_v)
    plsc.subcore_barrier()

    def chunk(j, carry):
        e0 = pl.multiple_of(wid * EP + j * CH, 8)
        pltpu.sync_copy(dst_hbm.at[pl.ds(e0, CH)], didx.at[0])
        pltpu.sync_copy(ones_v, acc.at[didx.at[0]], add=True)
        return carry

    lax.fori_loop(0, NCHUNK, chunk, 0)
    plsc.subcore_barrier()
    pltpu.sync_copy(acc.at[pl.ds(s * RP, RP)], stage)
    pltpu.sync_copy(stage, out_hbm.at[pl.ds(c * N + s * RP, RP)])


@functools.partial(
    pl.kernel,
    out_type=jax.ShapeDtypeStruct((NC * N, D), jnp.float32),
    mesh=_sc_mesh,
    scratch_types=[
        pltpu.VMEM((RP, D), jnp.float32),
        pltpu.VMEM((CH, D), jnp.float32),
        pltpu.VMEM((CH,), jnp.int32),
        pltpu.VMEM((1, CH), jnp.int32),
        pltpu.SemaphoreType.DMA,
        pltpu.VMEM_SHARED((N, D), jnp.float32),
    ],
)
def _sc_prop(t_hbm, src_hbm, dst_hbm, out_hbm, stage, rows, sidx, didx, gsem, acc):
    c = lax.axis_index("c")
    s = lax.axis_index("s")
    wid = c * NS + s
    pltpu.sync_copy(t_hbm.at[pl.ds(s * RP, RP)], stage)
    pltpu.sync_copy(stage, acc.at[pl.ds(s * RP, RP)])
    plsc.subcore_barrier()

    def chunk(j, carry):
        e0 = pl.multiple_of(wid * EP + j * CH, 8)
        pltpu.sync_copy(src_hbm.at[pl.ds(e0, CH)], sidx)
        pltpu.sync_copy(dst_hbm.at[pl.ds(e0, CH)], didx.at[0])
        pltpu.async_copy(t_hbm.at[sidx], rows, gsem).wait()
        pltpu.sync_copy(rows, acc.at[didx.at[0]], add=True)
        return carry

    lax.fori_loop(0, NCHUNK, chunk, 0)
    plsc.subcore_barrier()
    pltpu.sync_copy(acc.at[pl.ds(s * RP, RP)], stage)
    pltpu.sync_copy(stage, out_hbm.at[pl.ds(c * N + s * RP, RP)])


BR = 1000


def _dis(dp_ref):
    deg = dp_ref[0, :, 0:1] + dp_ref[1, :, 0:1] + 1.0
    return lax.rsqrt(deg)


def _tc1_body(dp_ref, x_ref, w_ref, t_ref):
    d = _dis(dp_ref)
    t_ref[...] = jnp.dot(x_ref[...], w_ref[...],
                         preferred_element_type=jnp.float32) * d


def _tc2_body(p_ref, t_ref, dp_ref, b_ref, u_ref):
    d = _dis(dp_ref)
    at = p_ref[0] + p_ref[1] - t_ref[...]
    h = jnp.maximum(d * at + b_ref[...], 0.0)
    u_ref[...] = d * h


def _tc3_body(q_ref, u_ref, dp_ref, wm_ref, bm_ref, wl_ref, bl_ref,
              mu_ref, lv_ref):
    d = _dis(dp_ref)
    hp = d * (q_ref[0] + q_ref[1] - u_ref[...])
    mu_ref[...] = jnp.dot(hp, wm_ref[...],
                          preferred_element_type=jnp.float32) + bm_ref[...]
    lv_ref[...] = jnp.dot(hp, wl_ref[...],
                          preferred_element_type=jnp.float32) + bl_ref[...]


def _row_spec(width):
    return pl.BlockSpec((BR, width), lambda i: (i, 0))


_dp_spec = pl.BlockSpec((2, BR, 8), lambda i: (0, i, 0))
_pq_spec = pl.BlockSpec((2, BR, D), lambda i: (0, i, 0))


def _full_spec(shape):
    nd = len(shape)
    return pl.BlockSpec(shape, lambda i: (0,) * nd)


_tc1 = pl.pallas_call(
    _tc1_body,
    grid=(N // BR,),
    in_specs=[_dp_spec, _row_spec(D), _full_spec((D, D))],
    out_specs=_row_spec(D),
    out_shape=jax.ShapeDtypeStruct((N, D), jnp.float32),
)

_tc2 = pl.pallas_call(
    _tc2_body,
    grid=(N // BR,),
    in_specs=[_pq_spec, _row_spec(D), _dp_spec, _full_spec((1, D))],
    out_specs=_row_spec(D),
    out_shape=jax.ShapeDtypeStruct((N, D), jnp.float32),
)

_tc3 = pl.pallas_call(
    _tc3_body,
    grid=(N // BR,),
    in_specs=[_pq_spec, _row_spec(D), _dp_spec,
              _full_spec((D, D_OUT)), _full_spec((1, D_OUT)),
              _full_spec((D, D_OUT)), _full_spec((1, D_OUT))],
    out_specs=[_row_spec(D_OUT), _row_spec(D_OUT)],
    out_shape=[jax.ShapeDtypeStruct((N, D_OUT), jnp.float32),
               jax.ShapeDtypeStruct((N, D_OUT), jnp.float32)],
)


@jax.jit
def kernel(x, edge_index, W1, b1, W_mu, b_mu, W_logvar, b_logvar):
    ei = edge_index.astype(jnp.int32)
    src = ei[0]
    dst = ei[1]
    zeros8 = jnp.zeros((RP, 8), jnp.float32)
    ones8 = jnp.ones((CH, 8), jnp.float32)

    deg_parts = _sc_degree(dst, zeros8, ones8).reshape(2, N, 8)
    t1 = _tc1(deg_parts, x, W1)
    p = _sc_prop(t1, src, dst).reshape(2, N, D)
    u = _tc2(p, t1, deg_parts, b1.reshape(1, D))
    q = _sc_prop(u, src, dst).reshape(2, N, D)
    mu, logvar = _tc3(q, u, deg_parts,
                      W_mu, b_mu.reshape(1, D_OUT),
                      W_logvar, b_logvar.reshape(1, D_OUT))
    return (mu, logvar)


# speedup vs baseline: 11.2795x; 11.2795x over previous
import functools

import jax
import jax.numpy as jnp
from jax import lax
from jax.experimental import pallas as pl
from jax.experimental.pallas import tpu as pltpu
from jax.experimental.pallas import tpu_sc as plsc

N = 10000
E = 320000
D = 128
D_OUT = 64

NC = 2
NS = 16
NW = NC * NS
CH = 80
EPW = E // NW
PN = 10240
RP = PN // NS

_sc_mesh = plsc.VectorSubcoreMesh(core_axis_name="c", subcore_axis_name="s")


@functools.partial(
    pl.kernel,
    out_type=jax.ShapeDtypeStruct((NC * PN, D), jnp.float32),
    mesh=_sc_mesh,
    scratch_types=[
        pltpu.VMEM((CH, D), jnp.float32),
        pltpu.VMEM((CH,), jnp.int32),
        pltpu.VMEM((1, CH), jnp.int32),
        pltpu.SemaphoreType.DMA,
        pltpu.VMEM_SHARED((PN, D), jnp.float32),
    ],
)
def _sc_prop(t_hbm, src_hbm, dst_hbm, out_hbm, rows, sidx, didx, gsem, acc):
    c = lax.axis_index("c")
    s = lax.axis_index("s")
    wid = c * NS + s

    def initc(j, carry):
        r0 = pl.multiple_of(s * RP + j * CH, 8)
        pltpu.sync_copy(t_hbm.at[pl.ds(r0, CH)], rows)
        pltpu.sync_copy(rows, acc.at[pl.ds(r0, CH)])
        return carry

    lax.fori_loop(0, RP // CH, initc, 0)
    plsc.subcore_barrier()

    def chunk(j, carry):
        e0 = pl.multiple_of(wid * EPW + j * CH, 8)
        pltpu.sync_copy(src_hbm.at[pl.ds(e0, CH)], sidx)
        pltpu.sync_copy(dst_hbm.at[pl.ds(e0, CH)], didx.at[0])
        pltpu.async_copy(t_hbm.at[sidx], rows, gsem).wait()
        pltpu.sync_copy(rows, acc.at[didx.at[0]], add=True)
        return carry

    lax.fori_loop(0, EPW // CH, chunk, 0)
    plsc.subcore_barrier()

    def wbc(j, carry):
        r0 = pl.multiple_of(s * RP + j * CH, 8)
        pltpu.sync_copy(acc.at[pl.ds(r0, CH)], rows)
        pltpu.sync_copy(rows, out_hbm.at[pl.ds(c * PN + r0, CH)])
        return carry

    lax.fori_loop(0, RP // CH, wbc, 0)


BR = 1000


def _tc0_body(x_ref, w_ref, xw_ref):
    xw_ref[...] = jnp.dot(x_ref[...], w_ref[...],
                          preferred_element_type=jnp.float32)


def _tc_mid_body(p_ref, t_ref, dis_ref, xw_ref, b_ref, fa_ref, fb_ref,
                 o_ref, dout_ref):
    comb = p_ref[0] + p_ref[1] - t_ref[...]
    is_first = fa_ref[...] > 0.0
    relu_on = fb_ref[...] > 0.0
    d = jnp.where(is_first[:, 0:1], lax.rsqrt(comb[:, 0:1]), dis_ref[...])
    zc = d * comb + b_ref[...]
    g = jnp.where(relu_on, jnp.maximum(zc, 0.0), zc)
    o_ref[...] = jnp.where(is_first, d * xw_ref[...],
                           jnp.where(relu_on, d * g, g))
    dout_ref[...] = d


def _tc3_body(hp_ref, wm_ref, bm_ref, wl_ref, bl_ref, mu_ref, lv_ref):
    hp = hp_ref[...]
    mu_ref[...] = jnp.dot(hp, wm_ref[...],
                          preferred_element_type=jnp.float32) + bm_ref[...]
    lv_ref[...] = jnp.dot(hp, wl_ref[...],
                          preferred_element_type=jnp.float32) + bl_ref[...]


def _row_spec(width):
    return pl.BlockSpec((BR, width), lambda i: (i, 0))


_pq_spec = pl.BlockSpec((2, BR, D), lambda i: (0, i, 0))


def _full_spec(shape):
    nd = len(shape)
    return pl.BlockSpec(shape, lambda i: (0,) * nd)


_tc0 = pl.pallas_call(
    _tc0_body,
    grid=(N // BR,),
    in_specs=[_row_spec(D), _full_spec((D, D))],
    out_specs=_row_spec(D),
    out_shape=jax.ShapeDtypeStruct((PN, D), jnp.float32),
)

_tc_mid = pl.pallas_call(
    _tc_mid_body,
    grid=(N // BR,),
    in_specs=[_pq_spec, _row_spec(D), _row_spec(1), _row_spec(D),
              _full_spec((1, D)), _full_spec((1, D)), _full_spec((1, D))],
    out_specs=[_row_spec(D), _row_spec(1)],
    out_shape=[jax.ShapeDtypeStruct((PN, D), jnp.float32),
               jax.ShapeDtypeStruct((PN, 1), jnp.float32)],
)

_tc3 = pl.pallas_call(
    _tc3_body,
    grid=(N // BR,),
    in_specs=[_row_spec(D),
              _full_spec((D, D_OUT)), _full_spec((1, D_OUT)),
              _full_spec((D, D_OUT)), _full_spec((1, D_OUT))],
    out_specs=[_row_spec(D_OUT), _row_spec(D_OUT)],
    out_shape=[jax.ShapeDtypeStruct((N, D_OUT), jnp.float32),
               jax.ShapeDtypeStruct((N, D_OUT), jnp.float32)],
)


@jax.jit
def kernel(x, edge_index, W1, b1, W_mu, b_mu, W_logvar, b_logvar):
    ei = edge_index.astype(jnp.int32)
    src = ei[0]
    dst = ei[1]

    xw = _tc0(x, W1)

    t0 = jnp.ones((PN, D), jnp.float32)
    dis0 = jnp.ones((PN, 1), jnp.float32)
    zerosD = jnp.zeros((1, D), jnp.float32)
    onesD = jnp.ones((1, D), jnp.float32)

    niter = 3 + lax.optimization_barrier(jnp.int32(0))

    def layer(i, carry):
        t, dis = carry
        fa = jnp.where(i == 0, onesD, zerosD)
        fb = jnp.where(i == 1, onesD, zerosD)
        b_i = jnp.where(i == 1, b1.reshape(1, D), zerosD)
        p = _sc_prop(t, src, dst).reshape(NC, PN, D)
        t2, dis2 = _tc_mid(p, t, dis, xw, b_i, fa, fb)
        return (t2, dis2)

    hp, _ = lax.fori_loop(0, niter, layer, (t0, dis0))

    mu, logvar = _tc3(hp,
                      W_mu, b_mu.reshape(1, D_OUT),
                      W_logvar, b_logvar.reshape(1, D_OUT))
    return (mu, logvar)
